# baseline (speedup 1.0000x reference)
import functools

import jax
import jax.numpy as jnp
from jax import lax
from jax.experimental import pallas as pl
from jax.experimental.pallas import tpu as pltpu
from jax.experimental.pallas import tpu_sc as plsc

NC = 2
NS = 16
NW = NC * NS

VBLK = 4096


def _transpose_block(tt_ref, out_ref):
    x = tt_ref[...]
    E = x.shape[0]
    half = jnp.broadcast_to(
        jnp.arange(0, 128, 2, dtype=jnp.int32)[None, :], (E, 64))
    for c in range(x.shape[1] // 128):
        xs = jax.lax.slice(x, (0, c * 128), (E, (c + 1) * 128))
        z = jnp.concatenate([jnp.take_along_axis(xs, half, axis=1),
                             jnp.take_along_axis(xs, half + 1, axis=1)],
                            axis=0)
        out_ref[pl.ds(c * 64, 64), :] = z.T


@jax.jit
def _tc_detranspose(table_t):
    E, V = table_t.shape
    grid = pl.cdiv(V, VBLK)
    return pl.pallas_call(
        _transpose_block,
        grid=(grid,),
        in_specs=[pl.BlockSpec((E, VBLK), lambda i: (0, i))],
        out_specs=pl.BlockSpec((VBLK // 2, 128), lambda i: (i, 0)),
        out_shape=jax.ShapeDtypeStruct((V // 2, 128), jnp.float32),
    )(table_t)


@jax.jit
def _sc_gather_t(table, idx_t):
    H, B = idx_t.shape
    D = table.shape[1]
    L = 128
    assert B == NW * L and D == 64
    mesh = plsc.VectorSubcoreMesh(core_axis_name="c", subcore_axis_name="s")

    @functools.partial(
        pl.kernel,
        mesh=mesh,
        out_type=jax.ShapeDtypeStruct((H, D // 8, NW, 8, L), jnp.float32),
        scratch_types=[
            pltpu.VMEM((H, L), jnp.int32),
            pltpu.VMEM((L, D), jnp.float32),
            pltpu.VMEM((L, D), jnp.float32),
            pltpu.VMEM((D // 8, 8, L), jnp.float32),
            pltpu.VMEM((D // 8, 8, L), jnp.float32),
            pltpu.SemaphoreType.DMA,
            pltpu.SemaphoreType.DMA,
            pltpu.SemaphoreType.DMA,
            pltpu.SemaphoreType.DMA,
        ],
        compiler_params=pltpu.CompilerParams(
            use_tc_tiling_on_sc=False, needs_layout_passes=False),
    )
    def k(table_hbm, idx_hbm, out_hbm, idx_v, rows0, rows1, ov0, ov1,
          g0, g1, o0, o1):
        wid = lax.axis_index("s") * NC + lax.axis_index("c")
        b0 = wid * L
        pltpu.sync_copy(idx_hbm.at[:, pl.ds(b0, L)], idx_v)

        rows = (rows0, rows1)
        ovs = (ov0, ov1)
        gsem = (g0, g1)
        osem = (o0, o1)
        lane = lax.iota(jnp.int32, 16)

        def g_start(h, p):
            pltpu.async_copy(table_hbm.at[idx_v.at[h]], rows[p], gsem[p])

        def g_wait(p):
            pltpu.make_async_copy(
                table_hbm.at[idx_v.at[0]], rows[p], gsem[p]).wait()

        def o_start(h, p):
            pltpu.async_copy(ovs[p], out_hbm.at[h, :, wid], osem[p])

        def o_wait(h, p):
            pltpu.make_async_copy(
                ovs[p], out_hbm.at[h, :, wid], osem[p]).wait()

        def transpose_step(p):
            r = rows[p]
            ov = ovs[p]

            def e_body(e, carry):
                col = jnp.broadcast_to(e, (16,)).astype(jnp.int32)
                ei = e // 8
                es = lax.rem(e, 8)
                for kk in range(L // 16):
                    v = plsc.load_gather(r, [lane + (16 * kk), col])
                    ov[ei, es, pl.ds(16 * kk, 16)] = v
                return carry

            lax.fori_loop(0, D, e_body, 0)

        g_start(0, 0)

        def body2(j, carry):
            h0 = 2 * j

            @pl.when(h0 + 1 < H)
            def _():
                g_start(h0 + 1, 1)
            g_wait(0)
            @pl.when(h0 >= 2)
            def _():
                o_wait(h0 - 2, 0)
            transpose_step(0)
            o_start(h0, 0)

            @pl.when(h0 + 2 < H)
            def _():
                g_start(h0 + 2, 0)
            g_wait(1)
            @pl.when(h0 >= 1)
            def _():
                o_wait(h0 - 1, 1)
            transpose_step(1)
            o_start(h0 + 1, 1)
            return carry

        lax.fori_loop(0, H // 2, body2, 0)
        o_wait(H - 2, 0)
        o_wait(H - 1, 1)

    return k(table, idx_t)


def kernel(x, input_embedding):
    V, D = input_embedding.shape
    table_pairs = _tc_detranspose(input_embedding.T)
    table_rm = table_pairs.reshape(V, D)
    out5 = _sc_gather_t(table_rm, x.T.astype(jnp.int32))
    return (out5.transpose(2, 4, 0, 1, 3)
            .reshape(x.shape[0], x.shape[1], D))


# speedup vs baseline: 1.2902x; 1.2902x over previous
import functools

import jax
import jax.numpy as jnp
from jax import lax
from jax.experimental import pallas as pl
from jax.experimental.pallas import tpu as pltpu
from jax.experimental.pallas import tpu_sc as plsc

NC = 2
NS = 16
NW = NC * NS


@functools.partial(jax.jit, static_argnums=(2, 3))
def _sc_gather(table, idx, chunk, b_per_w):
    B = idx.shape[0]
    D = table.shape[1]
    n_chunks = b_per_w // chunk
    assert n_chunks * chunk == b_per_w and n_chunks % 2 == 0
    pairs = n_chunks // 2
    mesh = plsc.VectorSubcoreMesh(core_axis_name="c", subcore_axis_name="s")

    @functools.partial(
        pl.kernel,
        mesh=mesh,
        out_type=jax.ShapeDtypeStruct((B, D), jnp.float32),
        scratch_types=[
            pltpu.VMEM((b_per_w,), jnp.int32),
            pltpu.VMEM((chunk, D), jnp.float32),
            pltpu.VMEM((chunk, D), jnp.float32),
            pltpu.SemaphoreType.DMA,
            pltpu.SemaphoreType.DMA,
            pltpu.SemaphoreType.DMA,
            pltpu.SemaphoreType.DMA,
        ],
        compiler_params=pltpu.CompilerParams(use_tc_tiling_on_sc=False),
    )
    def k(table_hbm, idx_hbm, out_hbm, idx_v, rows0, rows1, g0, g1, o0, o1):
        wid = lax.axis_index("s") * NC + lax.axis_index("c")
        w_base = wid * b_per_w
        pltpu.sync_copy(idx_hbm.at[pl.ds(w_base, b_per_w)], idx_v)

        def g_start(c, rows, sem):
            pltpu.async_copy(
                table_hbm.at[idx_v.at[pl.ds(c * chunk, chunk)]], rows, sem)

        def g_wait(rows, sem):
            pltpu.make_async_copy(
                table_hbm.at[idx_v.at[pl.ds(0, chunk)]], rows, sem).wait()

        def o_start(c, rows, sem):
            pltpu.async_copy(
                rows, out_hbm.at[pl.ds(w_base + c * chunk, chunk)], sem)

        def o_wait(c, rows, sem):
            pltpu.make_async_copy(
                rows, out_hbm.at[pl.ds(w_base + c * chunk, chunk)], sem).wait()

        g_start(0, rows0, g0)
        g_start(1, rows1, g1)

        def body(j, carry):
            c = 2 * j
            g_wait(rows0, g0)
            o_start(c, rows0, o0)
            g_wait(rows1, g1)
            o_start(c + 1, rows1, o1)
            o_wait(c, rows0, o0)
            g_start(c + 2, rows0, g0)
            o_wait(c + 1, rows1, o1)
            g_start(c + 3, rows1, g1)
            return carry

        lax.fori_loop(0, pairs - 1, body, 0)

        c = n_chunks - 2
        g_wait(rows0, g0)
        o_start(c, rows0, o0)
        g_wait(rows1, g1)
        o_start(c + 1, rows1, o1)
        o_wait(c, rows0, o0)
        o_wait(c + 1, rows1, o1)

    return k(table, idx)


def kernel(x, input_embedding):
    B = x.shape[0] * x.shape[1]
    D = input_embedding.shape[1]
    idx = x.reshape(B).astype(jnp.int32)
    b_per_w = B // NW
    out = _sc_gather(input_embedding, idx, 640, b_per_w)
    return out.reshape(x.shape + (D,))
